# baseline (speedup 1.0000x reference)
import functools

import jax
import jax.numpy as jnp
from jax import lax
from jax.experimental import pallas as pl
from jax.experimental.pallas import tpu as pltpu
from jax.experimental.pallas import tpu_sc as plsc

NUM_ROWS = 100001
DIM = 32
BATCH = 16384
LANES = 16
OCHUNK = 4096
N_OCHUNK = BATCH // OCHUNK


def kernel(idx, table):
    info = plsc.get_sparse_core_info()
    num_cores, num_subcores = info.num_cores, info.num_subcores
    num_workers = num_cores * num_subcores
    assert num_workers == DIM

    mesh = plsc.VectorSubcoreMesh(core_axis_name="c", subcore_axis_name="s")

    @functools.partial(
        pl.kernel,
        mesh=mesh,
        out_type=jax.ShapeDtypeStruct((DIM, BATCH), jnp.float32),
        scratch_types=[
            pltpu.VMEM((1, NUM_ROWS), jnp.float32),
            pltpu.VMEM((BATCH,), jnp.int32),
            pltpu.VMEM((1, OCHUNK), jnp.float32),
            pltpu.VMEM((1, OCHUNK), jnp.float32),
            pltpu.SemaphoreType.DMA,
            pltpu.SemaphoreType.DMA,
            pltpu.SemaphoreType.DMA,
        ],
        compiler_params=pltpu.CompilerParams(needs_layout_passes=False),
    )
    def gather_kernel(
        tab_t, idx_hbm, out_t, col_v, idx_v, out_a, out_b, sem_c, sem_i, sem_o
    ):
        wid = lax.axis_index("s") * num_cores + lax.axis_index("c")
        col_dma = pltpu.async_copy(tab_t.at[pl.ds(wid, 1), :], col_v, sem_c)
        idx_dma = pltpu.async_copy(idx_hbm, idx_v, sem_i)
        idx_dma.wait()
        col_dma.wait()
        zeros = jnp.zeros((LANES,), jnp.int32)
        bufs = (out_a, out_b)
        out_dmas = [None, None]

        for chunk in range(N_OCHUNK):
            buf = bufs[chunk % 2]
            if out_dmas[chunk % 2] is not None:
                out_dmas[chunk % 2].wait()
            base = chunk * OCHUNK

            @plsc.parallel_loop(0, OCHUNK // LANES, unroll=4)
            def gather_group(g):
                iv = idx_v[pl.ds(base + g * LANES, LANES)]
                vals = plsc.load_gather(col_v, [zeros, iv])
                buf[0, pl.ds(g * LANES, LANES)] = vals
            out_dmas[chunk % 2] = pltpu.async_copy(
                buf, out_t.at[pl.ds(wid, 1), pl.ds(base, OCHUNK)], sem_o
            )

        out_dmas[0].wait()
        out_dmas[1].wait()

    return gather_kernel(table.T, idx).T


# speedup vs baseline: 1.0086x; 1.0086x over previous
import functools

import jax
import jax.numpy as jnp
from jax import lax
from jax.experimental import pallas as pl
from jax.experimental.pallas import tpu as pltpu
from jax.experimental.pallas import tpu_sc as plsc

NUM_ROWS = 100001
DIM = 32
BATCH = 16384
LANES = 16
OCHUNK = 4096
N_OCHUNK = BATCH // OCHUNK


def kernel(idx, table):
    info = plsc.get_sparse_core_info()
    num_cores, num_subcores = info.num_cores, info.num_subcores
    num_workers = num_cores * num_subcores
    assert num_workers == DIM

    mesh = plsc.VectorSubcoreMesh(core_axis_name="c", subcore_axis_name="s")

    @functools.partial(
        pl.kernel,
        mesh=mesh,
        out_type=jax.ShapeDtypeStruct((DIM, BATCH), jnp.float32),
        scratch_types=[
            pltpu.VMEM((1, NUM_ROWS), jnp.float32),
            pltpu.VMEM((BATCH,), jnp.int32),
            pltpu.VMEM((1, OCHUNK), jnp.float32),
            pltpu.VMEM((1, OCHUNK), jnp.float32),
            pltpu.SemaphoreType.DMA,
            pltpu.SemaphoreType.DMA,
            pltpu.SemaphoreType.DMA,
        ],
        compiler_params=pltpu.CompilerParams(needs_layout_passes=False),
    )
    def gather_kernel(
        tab_t, idx_hbm, out_t, col_v, idx_v, out_a, out_b, sem_c, sem_i, sem_o
    ):
        wid = lax.axis_index("s") * num_cores + lax.axis_index("c")
        col_dma = pltpu.async_copy(tab_t.at[pl.ds(wid, 1), :], col_v, sem_c)
        idx_dma = pltpu.async_copy(idx_hbm, idx_v, sem_i)
        idx_dma.wait()
        col_dma.wait()
        zeros = jnp.zeros((LANES,), jnp.int32)
        bufs = (out_a, out_b)
        out_dmas = [None, None]

        for chunk in range(N_OCHUNK):
            buf = bufs[chunk % 2]
            if out_dmas[chunk % 2] is not None:
                out_dmas[chunk % 2].wait()
            base = chunk * OCHUNK

            @plsc.parallel_loop(0, OCHUNK // LANES, unroll=8)
            def gather_group(g):
                iv = idx_v[pl.ds(base + g * LANES, LANES)]
                vals = plsc.load_gather(col_v, [zeros, iv])
                buf[0, pl.ds(g * LANES, LANES)] = vals
            out_dmas[chunk % 2] = pltpu.async_copy(
                buf, out_t.at[pl.ds(wid, 1), pl.ds(base, OCHUNK)], sem_o
            )

        out_dmas[0].wait()
        out_dmas[1].wait()

    return gather_kernel(table.T, idx).T
